# Initial kernel scaffold; baseline (speedup 1.0000x reference)
import jax
import jax.numpy as jnp
from jax.experimental import pallas as pl


def kernel(x, edge_index, Wsrc1, Wdst1, attn1, b1, Wsrc2, Wdst2, attn2, b2):
    raise NotImplementedError("write your pallas kernel here")


# speedup vs baseline: 31.3005x; 31.3005x over previous
import functools

import jax
import jax.numpy as jnp
from jax import lax
from jax.experimental import pallas as pl
from jax.experimental.pallas import tpu as pltpu
from jax.experimental.pallas import tpu_sc as plsc

F = 128
NC, NS = 2, 16
NW = NC * NS
C = 80
GRP = C // 16


def _sc_gat_layer(fs, fd, src, dst, attn_flat, num_heads):
    n = fs.shape[0]
    e_total = src.shape[0]
    epw = e_total // NW
    nchunk = epw // C
    assert epw * NW == e_total and nchunk * C == epw
    n_pad = -(-n // 128) * 128
    rows_per_tile = n_pad // NS
    jh = 8 // num_heads
    rows_den = n_pad // 8
    den_chunks8 = rows_den // 8

    zeros = jnp.zeros((n_pad, F), jnp.float32)
    mesh = plsc.VectorSubcoreMesh(core_axis_name="c", subcore_axis_name="s")

    @functools.partial(
        pl.kernel,
        mesh=mesh,
        compiler_params=pltpu.CompilerParams(needs_layout_passes=False),
        out_type=(
            jax.ShapeDtypeStruct((NC, n_pad, F), jnp.float32),
            jax.ShapeDtypeStruct((NC, rows_den, F), jnp.float32),
        ),
        scratch_types=[
            pltpu.VMEM((C,), jnp.int32),
            pltpu.VMEM((C,), jnp.int32),
            pltpu.VMEM((C,), jnp.int32),
            pltpu.VMEM((C, F), jnp.float32),
            pltpu.VMEM((C, F), jnp.float32),
            pltpu.VMEM((C, F), jnp.float32),
            pltpu.VMEM((F,), jnp.float32),
            pltpu.VMEM_SHARED((n_pad, F), jnp.float32),
            pltpu.VMEM_SHARED((rows_den, F), jnp.float32),
            pltpu.SemaphoreType.DMA,
            pltpu.SemaphoreType.DMA,
        ],
    )
    def k(fs_hbm, fd_hbm, src_hbm, dst_hbm, attn_hbm, zeros_hbm,
          out_hbm, outden_hbm,
          sidx, didx, didx8, fsb, fdb, denmsg, attnv,
          acc, accden, sem1, sem2):
        cid = lax.axis_index("c")
        sid = lax.axis_index("s")
        wid = sid * NC + cid
        base = wid * epw

        pltpu.sync_copy(attn_hbm, attnv)
        attn_vecs = [attnv[pl.ds(16 * j, 16)] for j in range(8)]
        lanes = jnp.arange(16, dtype=jnp.int32)
        onehots = [lanes == h for h in range(num_heads)]
        perms = [jnp.bitwise_xor(lanes, sh)[:, None] for sh in (8, 4, 2, 1)]
        dnums = lax.GatherDimensionNumbers(
            offset_dims=(), collapsed_slice_dims=(0,), start_index_map=(0,))

        def vgather(v, p):
            return lax.gather(v, p, dnums, (1,),
                              mode=lax.GatherScatterMode.PROMISE_IN_BOUNDS)

        def xlane_sum(v):
            for p in perms:
                v = v + vgather(v, p)
            return v

        r0 = sid * rows_per_tile
        pltpu.sync_copy(zeros_hbm.at[pl.ds(r0, rows_per_tile)],
                        acc.at[pl.ds(r0, rows_per_tile)])
        for t in range(-(-den_chunks8 // NS)):
            ch = sid + NS * t

            @pl.when(ch < den_chunks8)
            def _():
                pltpu.sync_copy(zeros_hbm.at[pl.ds(ch * 8, 8)],
                                accden.at[pl.ds(ch * 8, 8)])

        plsc.subcore_barrier()

        def chunk_body(kk, carry):
            b = base + kk * C
            pltpu.sync_copy(src_hbm.at[pl.ds(b, C)], sidx)
            pltpu.sync_copy(dst_hbm.at[pl.ds(b, C)], didx)
            cp1 = pltpu.async_copy(fs_hbm.at[sidx], fsb, sem1)
            cp2 = pltpu.async_copy(fd_hbm.at[didx], fdb, sem2)
            cp1.wait()
            cp2.wait()

            def group_body(g, carry2):
                dv = didx[pl.ds(g * 16, 16)]
                didx8[pl.ds(g * 16, 16)] = dv >> 3
                dv7 = dv & 7
                for l in range(16):
                    e = g * 16 + l
                    fsv = [fsb[e, pl.ds(16 * j, 16)] for j in range(8)]
                    fdv = [fdb[e, pl.ds(16 * j, 16)] for j in range(8)]
                    qs = []
                    for h in range(num_heads):
                        t = None
                        for j in range(jh * h, jh * (h + 1)):
                            z = fsv[j] + fdv[j]
                            lr = jnp.maximum(z, 0.2 * z)
                            tj = lr * attn_vecs[j]
                            t = tj if t is None else t + tj
                        q = jnp.exp(xlane_sum(t))
                        qs.append(q)
                    for j in range(8):
                        fsb[e, pl.ds(16 * j, 16)] = fsv[j] * qs[j // jh]
                    if num_heads == 1:
                        den_row = qs[0]
                    else:
                        den_row = jnp.where(onehots[0], qs[0], 0.0)
                        for h in range(1, num_heads):
                            den_row = den_row + jnp.where(
                                onehots[h], qs[h], 0.0)
                    dm = vgather(dv7, jnp.full((16, 1), l, jnp.int32))
                    for j in range(8):
                        denmsg[e, pl.ds(16 * j, 16)] = jnp.where(
                            dm == j, den_row, 0.0)
                return carry2

            lax.fori_loop(0, GRP, group_body, 0)
            pltpu.sync_copy(fsb, acc.at[didx], add=True)
            pltpu.sync_copy(denmsg, accden.at[didx8], add=True)
            return carry

        lax.fori_loop(0, nchunk, chunk_body, 0)
        plsc.subcore_barrier()

        pltpu.sync_copy(acc.at[pl.ds(r0, rows_per_tile)],
                        out_hbm.at[cid, pl.ds(r0, rows_per_tile)])
        for t in range(-(-den_chunks8 // NS)):
            ch = sid + NS * t

            @pl.when(ch < den_chunks8)
            def _():
                pltpu.sync_copy(
                    accden.at[pl.ds(ch * 8, 8)],
                    outden_hbm.at[cid, pl.ds(ch * 8, 8)])

    acc, den = k(fs, fd, src, dst, attn_flat, zeros)
    den = den.reshape(NC, n_pad, 16)[:, :, :num_heads]
    return acc[:, :n, :], den[:, :n, :]


def _proj_kernel(x_ref, w1_ref, w2_ref, o1_ref, o2_ref):
    o1_ref[...] = jnp.dot(x_ref[...], w1_ref[...],
                          preferred_element_type=jnp.float32)
    o2_ref[...] = jnp.dot(x_ref[...], w2_ref[...],
                          preferred_element_type=jnp.float32)


def _proj2(x, w1, w2, blk):
    n = x.shape[0]
    grid = n // blk
    return pl.pallas_call(
        _proj_kernel,
        grid=(grid,),
        in_specs=[
            pl.BlockSpec((blk, x.shape[1]), lambda i: (i, 0)),
            pl.BlockSpec(w1.shape, lambda i: (0, 0)),
            pl.BlockSpec(w2.shape, lambda i: (0, 0)),
        ],
        out_specs=[
            pl.BlockSpec((blk, w1.shape[1]), lambda i: (i, 0)),
            pl.BlockSpec((blk, w2.shape[1]), lambda i: (i, 0)),
        ],
        out_shape=[
            jax.ShapeDtypeStruct((n, w1.shape[1]), jnp.float32),
            jax.ShapeDtypeStruct((n, w2.shape[1]), jnp.float32),
        ],
    )(x, w1, w2)


def _mid_kernel(a0_ref, a1_ref, d0_ref, d1_ref, bsel_ref, b1_ref,
                w1_ref, w2_ref, o1_ref, o2_ref):
    s = a0_ref[...] + a1_ref[...]
    den = jnp.dot(d0_ref[...] + d1_ref[...], bsel_ref[...],
                  preferred_element_type=jnp.float32)
    h = jnp.maximum(s / (den + 1e-9) + b1_ref[...], 0.0)
    o1_ref[...] = jnp.dot(h, w1_ref[...], preferred_element_type=jnp.float32)
    o2_ref[...] = jnp.dot(h, w2_ref[...], preferred_element_type=jnp.float32)


def _mid(a0, a1, d0, d1, bsel, b1, w1, w2, blk):
    n = a0.shape[0]
    nh = d0.shape[1]
    grid = n // blk
    return pl.pallas_call(
        _mid_kernel,
        grid=(grid,),
        in_specs=[
            pl.BlockSpec((blk, F), lambda i: (i, 0)),
            pl.BlockSpec((blk, F), lambda i: (i, 0)),
            pl.BlockSpec((blk, nh), lambda i: (i, 0)),
            pl.BlockSpec((blk, nh), lambda i: (i, 0)),
            pl.BlockSpec(bsel.shape, lambda i: (0, 0)),
            pl.BlockSpec(b1.shape, lambda i: (0, 0)),
            pl.BlockSpec(w1.shape, lambda i: (0, 0)),
            pl.BlockSpec(w2.shape, lambda i: (0, 0)),
        ],
        out_specs=[
            pl.BlockSpec((blk, F), lambda i: (i, 0)),
            pl.BlockSpec((blk, F), lambda i: (i, 0)),
        ],
        out_shape=[
            jax.ShapeDtypeStruct((n, F), jnp.float32),
            jax.ShapeDtypeStruct((n, F), jnp.float32),
        ],
    )(a0, a1, d0, d1, bsel, b1, w1, w2)


def _fin_kernel(a0_ref, a1_ref, d0_ref, d1_ref, bsel_ref, b2_ref, o_ref):
    s = a0_ref[...] + a1_ref[...]
    den = jnp.dot(d0_ref[...] + d1_ref[...], bsel_ref[...],
                  preferred_element_type=jnp.float32)
    o_ref[...] = s / (den + 1e-9) + b2_ref[...]


def _fin(a0, a1, d0, d1, bsel, b2, blk):
    n = a0.shape[0]
    nh = d0.shape[1]
    grid = n // blk
    return pl.pallas_call(
        _fin_kernel,
        grid=(grid,),
        in_specs=[
            pl.BlockSpec((blk, F), lambda i: (i, 0)),
            pl.BlockSpec((blk, F), lambda i: (i, 0)),
            pl.BlockSpec((blk, nh), lambda i: (i, 0)),
            pl.BlockSpec((blk, nh), lambda i: (i, 0)),
            pl.BlockSpec(bsel.shape, lambda i: (0, 0)),
            pl.BlockSpec(b2.shape, lambda i: (0, 0)),
        ],
        out_specs=pl.BlockSpec((blk, F), lambda i: (i, 0)),
        out_shape=jax.ShapeDtypeStruct((n, F), jnp.float32),
    )(a0, a1, d0, d1, bsel, b2)


def _head_select(num_heads):
    d = F // num_heads
    m = jnp.zeros((num_heads, F), jnp.float32)
    for h in range(num_heads):
        m = m.at[h, h * d:(h + 1) * d].set(1.0)
    return m


def kernel(x, edge_index, Wsrc1, Wdst1, attn1, b1, Wsrc2, Wdst2, attn2, b2):
    n = x.shape[0]
    src = edge_index[0]
    dst = edge_index[1]
    blk = 1000 if n % 1000 == 0 else 8

    fs1, fd1 = _proj2(x, Wsrc1, Wdst1, blk)
    acc1, den1 = _sc_gat_layer(fs1, fd1, src, dst, attn1.reshape(-1), 4)
    fs2, fd2 = _mid(acc1[0], acc1[1], den1[0], den1[1], _head_select(4),
                    b1.reshape(1, F), Wsrc2, Wdst2, blk)
    acc2, den2 = _sc_gat_layer(fs2, fd2, src, dst, attn2.reshape(-1), 1)
    return _fin(acc2[0], acc2[1], den2[0], den2[1], _head_select(1),
                b2.reshape(1, F), blk)


# speedup vs baseline: 48.8257x; 1.5599x over previous
import functools

import jax
import jax.numpy as jnp
from jax import lax
from jax.experimental import pallas as pl
from jax.experimental.pallas import tpu as pltpu
from jax.experimental.pallas import tpu_sc as plsc

F = 128
NC, NS = 2, 16
NW = NC * NS
C = 80
GRP = C // 16


def _sc_gat_layer(fs, fd, src, dst, attn_flat, num_heads):
    n = fs.shape[0]
    e_total = src.shape[0]
    epw = e_total // NW
    nchunk = epw // C
    nsuper = nchunk // 5
    assert epw * NW == e_total and nchunk * C == epw and nsuper * 5 == nchunk
    n_pad = -(-n // 128) * 128
    rows_per_tile = n_pad // NS
    jh = 8 // num_heads
    rows_den = n_pad // 16
    den_chunks8 = rows_den // 8

    zeros = jnp.zeros((n_pad, F), jnp.float32)
    mesh = plsc.VectorSubcoreMesh(core_axis_name="c", subcore_axis_name="s")

    @functools.partial(
        pl.kernel,
        mesh=mesh,
        compiler_params=pltpu.CompilerParams(needs_layout_passes=False),
        out_type=(
            jax.ShapeDtypeStruct((NC, n_pad, F), jnp.float32),
            jax.ShapeDtypeStruct((NC, rows_den, F), jnp.float32),
        ),
        scratch_types=[
            pltpu.VMEM((2, C), jnp.int32),
            pltpu.VMEM((2, C), jnp.int32),
            pltpu.VMEM((C,), jnp.int32),
            pltpu.VMEM((C,), jnp.int32),
            pltpu.VMEM((C,), jnp.int32),
            pltpu.VMEM((C,), jnp.int32),
            pltpu.VMEM((C, F), jnp.float32),
            pltpu.VMEM((C, F), jnp.float32),
            pltpu.VMEM((C, F), jnp.float32),
            pltpu.VMEM((C, F), jnp.float32),
            pltpu.VMEM((F,), jnp.float32),
            pltpu.VMEM_SHARED((n_pad, F), jnp.float32),
            pltpu.VMEM_SHARED((rows_den, F), jnp.float32),
            pltpu.SemaphoreType.DMA,
            pltpu.SemaphoreType.DMA,
            pltpu.SemaphoreType.DMA,
            pltpu.SemaphoreType.DMA,
            pltpu.SemaphoreType.DMA,
            pltpu.SemaphoreType.DMA,
            pltpu.SemaphoreType.DMA,
        ],
    )
    def k(fs_hbm, fd_hbm, src_hbm, dst_hbm, attn_hbm, zeros_hbm,
          out_hbm, outden_hbm,
          ix0, ix1, dd0, dd1, dr0, dr1, fb0, fb1, fdb, denmsg, attnv,
          acc, accden,
          semI, semF0, semF1, semFD, semM0, semM1, semD):
        idx2 = [ix0, ix1]
        didx = [dd0, dd1]
        didx16 = [dr0, dr1]
        fsb = [fb0, fb1]
        semF = [semF0, semF1]
        semM = [semM0, semM1]
        cid = lax.axis_index("c")
        sid = lax.axis_index("s")
        wid = sid * NC + cid
        base = wid * epw

        pltpu.sync_copy(attn_hbm, attnv)
        attn_vecs = [attnv[pl.ds(16 * j, 16)] for j in range(8)]
        lanes = jnp.arange(16, dtype=jnp.int32)
        onehots = [lanes == h for h in range(num_heads)]
        shift8 = ((lanes - 8) & 15)[:, None]
        dnums = lax.GatherDimensionNumbers(
            offset_dims=(), collapsed_slice_dims=(0,), start_index_map=(0,))

        def vgather(v, p):
            return lax.gather(v, p, dnums, (1,),
                              mode=lax.GatherScatterMode.PROMISE_IN_BOUNDS)

        r0 = sid * rows_per_tile
        pltpu.sync_copy(zeros_hbm.at[pl.ds(r0, rows_per_tile)],
                        acc.at[pl.ds(r0, rows_per_tile)])
        for t in range(-(-den_chunks8 // NS)):
            ch = sid + NS * t

            @pl.when(ch < den_chunks8)
            def _():
                pltpu.sync_copy(zeros_hbm.at[pl.ds(ch * 8, 8)],
                                accden.at[pl.ds(ch * 8, 8)])

        plsc.subcore_barrier()

        def wait_msg_scatter(p):
            pltpu.make_async_copy(fsb[p], acc.at[didx[p]], semM[p]).wait()

        def wait_den_scatter():
            pltpu.make_async_copy(denmsg, accden.at[didx16[0]], semD).wait()

        def compute_chunk(p):
            def group_body(g, carry2):
                dv = idx2[p][1, pl.ds(g * 16, 16)]
                didx[p][pl.ds(g * 16, 16)] = dv
                didx16[p][pl.ds(g * 16, 16)] = dv >> 4
                dmv = dv & 15
                for l in range(16):
                    e = g * 16 + l
                    fsv = [fsb[p][e, pl.ds(16 * j, 16)] for j in range(8)]
                    fdv = [fdb[e, pl.ds(16 * j, 16)] for j in range(8)]
                    qs = []
                    for h in range(num_heads):
                        t = None
                        for j in range(jh * h, jh * (h + 1)):
                            z = fsv[j] + fdv[j]
                            lr = jnp.maximum(z, 0.2 * z)
                            tj = lr * attn_vecs[j]
                            t = tj if t is None else t + tj
                        q = jnp.exp(jnp.full((16,), jnp.sum(t), jnp.float32))
                        qs.append(q)
                    for j in range(8):
                        fsb[p][e, pl.ds(16 * j, 16)] = fsv[j] * qs[j // jh]
                    den_row = jnp.where(onehots[0], qs[0], 0.0)
                    for h in range(1, num_heads):
                        den_row = den_row + jnp.where(onehots[h], qs[h], 0.0)
                    dm = vgather(dmv, jnp.full((16, 1), l, jnp.int32))
                    dsel = jnp.where((dm & 1) == 1,
                                     vgather(den_row, shift8), den_row)
                    g8 = dm >> 1
                    for j in range(8):
                        denmsg[e, pl.ds(16 * j, 16)] = jnp.where(
                            g8 == j, dsel, 0.0)
                return carry2

            lax.fori_loop(0, GRP, group_body, 0)

        def superchunk(s, carry):
            bs = base + s * (5 * C)
            for j in range(5):
                p = j & 1
                bj = bs + j * C
                if j == 0:
                    pltpu.sync_copy(src_hbm.at[pl.ds(bj, C)], idx2[0].at[0])
                    pltpu.sync_copy(dst_hbm.at[pl.ds(bj, C)], idx2[0].at[1])

                    @pl.when(s > 0)
                    def _():
                        wait_msg_scatter(0)
                        wait_den_scatter()
                    fs_d = pltpu.async_copy(
                        fs_hbm.at[idx2[0].at[0]], fsb[0], semF[0])
                    fd_d = pltpu.async_copy(
                        fd_hbm.at[idx2[0].at[1]], fdb, semFD)
                fs_d.wait()
                if j < 4:
                    ix_d1 = pltpu.async_copy(
                        src_hbm.at[pl.ds(bj + C, C)], idx2[1 - p].at[0], semI)
                    ix_d2 = pltpu.async_copy(
                        dst_hbm.at[pl.ds(bj + C, C)], idx2[1 - p].at[1], semI)
                fd_d.wait()
                if j < 4:
                    ix_d1.wait()
                    ix_d2.wait()
                    if j == 0:
                        @pl.when(s > 0)
                        def _():
                            wait_msg_scatter(1)
                    else:
                        msg_d.wait()
                    fs_d = pltpu.async_copy(
                        fs_hbm.at[idx2[1 - p].at[0]], fsb[1 - p],
                        semF[1 - p])
                if j > 0:
                    den_d.wait()
                compute_chunk(p)
                msg_d = pltpu.async_copy(
                    fsb[p], acc.at[didx[p]], semM[p], add=True)
                den_d = pltpu.async_copy(
                    denmsg, accden.at[didx16[p]], semD, add=True)
                if j < 4:
                    fd_d = pltpu.async_copy(
                        fd_hbm.at[idx2[1 - p].at[1]], fdb, semFD)
            return carry

        lax.fori_loop(0, nsuper, superchunk, 0)
        wait_msg_scatter(1)
        wait_msg_scatter(0)
        wait_den_scatter()
        plsc.subcore_barrier()

        pltpu.sync_copy(acc.at[pl.ds(r0, rows_per_tile)],
                        out_hbm.at[cid, pl.ds(r0, rows_per_tile)])
        for t in range(-(-den_chunks8 // NS)):
            ch = sid + NS * t

            @pl.when(ch < den_chunks8)
            def _():
                pltpu.sync_copy(
                    accden.at[pl.ds(ch * 8, 8)],
                    outden_hbm.at[cid, pl.ds(ch * 8, 8)])

    acc, den = k(fs, fd, src, dst, attn_flat, zeros)
    den = den.reshape(NC, n_pad, 8)[:, :, :num_heads]
    return acc[:, :n, :], den[:, :n, :]


def _proj_kernel(x_ref, w1_ref, w2_ref, o1_ref, o2_ref):
    o1_ref[...] = jnp.dot(x_ref[...], w1_ref[...],
                          preferred_element_type=jnp.float32)
    o2_ref[...] = jnp.dot(x_ref[...], w2_ref[...],
                          preferred_element_type=jnp.float32)


def _proj2(x, w1, w2, blk):
    n = x.shape[0]
    grid = n // blk
    return pl.pallas_call(
        _proj_kernel,
        grid=(grid,),
        in_specs=[
            pl.BlockSpec((blk, x.shape[1]), lambda i: (i, 0)),
            pl.BlockSpec(w1.shape, lambda i: (0, 0)),
            pl.BlockSpec(w2.shape, lambda i: (0, 0)),
        ],
        out_specs=[
            pl.BlockSpec((blk, w1.shape[1]), lambda i: (i, 0)),
            pl.BlockSpec((blk, w2.shape[1]), lambda i: (i, 0)),
        ],
        out_shape=[
            jax.ShapeDtypeStruct((n, w1.shape[1]), jnp.float32),
            jax.ShapeDtypeStruct((n, w2.shape[1]), jnp.float32),
        ],
    )(x, w1, w2)


def _mid_kernel(a0_ref, a1_ref, d0_ref, d1_ref, bsel_ref, b1_ref,
                w1_ref, w2_ref, o1_ref, o2_ref):
    s = a0_ref[...] + a1_ref[...]
    den = jnp.dot(d0_ref[...] + d1_ref[...], bsel_ref[...],
                  preferred_element_type=jnp.float32)
    h = jnp.maximum(s / (den + 1e-9) + b1_ref[...], 0.0)
    o1_ref[...] = jnp.dot(h, w1_ref[...], preferred_element_type=jnp.float32)
    o2_ref[...] = jnp.dot(h, w2_ref[...], preferred_element_type=jnp.float32)


def _mid(a0, a1, d0, d1, bsel, b1, w1, w2, blk):
    n = a0.shape[0]
    nh = d0.shape[1]
    grid = n // blk
    return pl.pallas_call(
        _mid_kernel,
        grid=(grid,),
        in_specs=[
            pl.BlockSpec((blk, F), lambda i: (i, 0)),
            pl.BlockSpec((blk, F), lambda i: (i, 0)),
            pl.BlockSpec((blk, nh), lambda i: (i, 0)),
            pl.BlockSpec((blk, nh), lambda i: (i, 0)),
            pl.BlockSpec(bsel.shape, lambda i: (0, 0)),
            pl.BlockSpec(b1.shape, lambda i: (0, 0)),
            pl.BlockSpec(w1.shape, lambda i: (0, 0)),
            pl.BlockSpec(w2.shape, lambda i: (0, 0)),
        ],
        out_specs=[
            pl.BlockSpec((blk, F), lambda i: (i, 0)),
            pl.BlockSpec((blk, F), lambda i: (i, 0)),
        ],
        out_shape=[
            jax.ShapeDtypeStruct((n, F), jnp.float32),
            jax.ShapeDtypeStruct((n, F), jnp.float32),
        ],
    )(a0, a1, d0, d1, bsel, b1, w1, w2)


def _fin_kernel(a0_ref, a1_ref, d0_ref, d1_ref, bsel_ref, b2_ref, o_ref):
    s = a0_ref[...] + a1_ref[...]
    den = jnp.dot(d0_ref[...] + d1_ref[...], bsel_ref[...],
                  preferred_element_type=jnp.float32)
    o_ref[...] = s / (den + 1e-9) + b2_ref[...]


def _fin(a0, a1, d0, d1, bsel, b2, blk):
    n = a0.shape[0]
    nh = d0.shape[1]
    grid = n // blk
    return pl.pallas_call(
        _fin_kernel,
        grid=(grid,),
        in_specs=[
            pl.BlockSpec((blk, F), lambda i: (i, 0)),
            pl.BlockSpec((blk, F), lambda i: (i, 0)),
            pl.BlockSpec((blk, nh), lambda i: (i, 0)),
            pl.BlockSpec((blk, nh), lambda i: (i, 0)),
            pl.BlockSpec(bsel.shape, lambda i: (0, 0)),
            pl.BlockSpec(b2.shape, lambda i: (0, 0)),
        ],
        out_specs=pl.BlockSpec((blk, F), lambda i: (i, 0)),
        out_shape=jax.ShapeDtypeStruct((n, F), jnp.float32),
    )(a0, a1, d0, d1, bsel, b2)


def _head_select(num_heads):
    d = F // num_heads
    m = jnp.zeros((num_heads, F), jnp.float32)
    for h in range(num_heads):
        m = m.at[h, h * d:(h + 1) * d].set(1.0)
    return m


def kernel(x, edge_index, Wsrc1, Wdst1, attn1, b1, Wsrc2, Wdst2, attn2, b2):
    n = x.shape[0]
    blk = 1000 if n % 1000 == 0 else 8

    src = edge_index[0]
    dst = edge_index[1]
    fs1, fd1 = _proj2(x, Wsrc1, Wdst1, blk)
    acc1, den1 = _sc_gat_layer(fs1, fd1, src, dst, attn1.reshape(-1), 4)
    fs2, fd2 = _mid(acc1[0], acc1[1], den1[0], den1[1], _head_select(4),
                    b1.reshape(1, F), Wsrc2, Wdst2, blk)
    acc2, den2 = _sc_gat_layer(fs2, fd2, src, dst, attn2.reshape(-1), 1)
    return _fin(acc2[0], acc2[1], den2[0], den2[1], _head_select(1),
                b2.reshape(1, F), blk)
